# baseline (speedup 1.0000x reference)
import functools
import math

import jax
import jax.numpy as jnp
from jax import lax
from jax.experimental import pallas as pl
from jax.experimental.pallas import tpu as pltpu
from jax.experimental.pallas import tpu_sc as plsc

N = 8192
K = 32
HID = 128
NG = 50
NGP = 64
NGRAPH = 256
CUTOFF = 10.0
MAX_Z = 100

RB = 128
CH = 1024
NCH = N // CH

NB = 128
E = N * K

INVALID_D = 20.0


def _k0_body(c0_ref, c1_ref, pos_r_ref, posc_ref, batch_r_ref, batch_c_ref,
             nbr_ref, d_ref, cw_ref, D_ref):
  b = pl.program_id(0)
  c0 = c0_ref[b]
  c1 = c1_ref[b]
  cb0 = c0 // CH
  cb1 = (c1 + CH - 1) // CH
  rbase = b * RB

  batch_r = batch_r_ref[...]
  row_ids = rbase + lax.broadcasted_iota(jnp.int32, (RB, CH), 0)
  lane_ids = lax.broadcasted_iota(jnp.int32, (RB, CH), 1)

  def fill(cb, _):
    acc = jnp.zeros((RB, CH), jnp.float32)
    for k in range(3):
      pr = pos_r_ref[:, k:k + 1]
      pc = posc_ref[k, cb]
      dk = pr - pc[None, :]
      acc = acc + dk * dk
    bc = batch_c_ref[cb]
    col_ids = cb * CH + lane_ids
    mask = (batch_r == bc[None, :]) & (col_ids != row_ids) & (acc < CUTOFF * CUTOFF)
    D_ref[cb] = jnp.where(mask, acc, jnp.inf)
    return 0

  lax.fori_loop(cb0, cb1, fill, 0)

  BIGI = jnp.int32(1 << 30)
  last_v = jnp.full((RB, 1), -jnp.inf, jnp.float32)
  last_i = jnp.full((RB, 1), -1, jnp.int32)

  for j in range(K):
    def scan_chunk(cb, carry):
      mv, mi = carry
      chunk = D_ref[cb]
      col_ids = cb * CH + lane_ids
      ok = (chunk > last_v) | ((chunk == last_v) & (col_ids > last_i))
      effv = jnp.where(ok, chunk, jnp.inf)
      effi = jnp.where(ok, col_ids, BIGI)
      vmin = jnp.min(effv, axis=1, keepdims=True)
      imin = jnp.min(jnp.where(effv == vmin, effi, BIGI), axis=1, keepdims=True)
      better = (vmin < mv) | ((vmin == mv) & (imin < mi))
      return (jnp.where(better, vmin, mv), jnp.where(better, imin, mi))

    mv0 = jnp.full((RB, 1), jnp.inf, jnp.float32)
    mi0 = jnp.full((RB, 1), BIGI, jnp.int32)
    mv, mi = lax.fori_loop(cb0, cb1, scan_chunk, (mv0, mi0))
    last_v, last_i = mv, mi

    valid = mv < jnp.float32(1e30)
    idx = jnp.where(valid, mi, 0)
    d = jnp.where(valid, jnp.sqrt(jnp.maximum(mv, 0.0)), INVALID_D)
    cw = jnp.where(valid, 0.5 * (jnp.cos(d * (math.pi / CUTOFF)) + 1.0), 0.0)
    nbr_ref[:, j:j + 1] = idx
    d_ref[:, j:j + 1] = d
    cw_ref[:, j:j + 1] = cw


def _neighbor_search(pos, batch, c0s, c1s):
  posc = pos.T.reshape(3, NCH, CH)
  batch_c = batch.reshape(NCH, CH)
  batch_r = batch.reshape(N, 1)
  grid = N // RB
  return pl.pallas_call(
      _k0_body,
      grid=(grid,),
      in_specs=[
          pl.BlockSpec(memory_space=pltpu.SMEM),
          pl.BlockSpec(memory_space=pltpu.SMEM),
          pl.BlockSpec((RB, 3), lambda b: (b, 0)),
          pl.BlockSpec((3, NCH, CH), lambda b: (0, 0, 0)),
          pl.BlockSpec((RB, 1), lambda b: (b, 0)),
          pl.BlockSpec((NCH, CH), lambda b: (0, 0)),
      ],
      out_specs=[
          pl.BlockSpec((RB, K), lambda b: (b, 0)),
          pl.BlockSpec((RB, K), lambda b: (b, 0)),
          pl.BlockSpec((RB, K), lambda b: (b, 0)),
      ],
      out_shape=[
          jax.ShapeDtypeStruct((N, K), jnp.int32),
          jax.ShapeDtypeStruct((N, K), jnp.float32),
          jax.ShapeDtypeStruct((N, K), jnp.float32),
      ],
      scratch_shapes=[pltpu.VMEM((NCH, RB, CH), jnp.float32)],
  )(c0s, c1s, pos, posc, batch_r, batch_c)


def _silu(x):
  return x * jax.nn.sigmoid(x)


def _t1_body(first, h_ref, a0_ref, a1_ref, d_ref, cw_ref, prevW_ref, prevb_ref,
             w1_ref, b1_ref, w2_ref, b2_ref, l1w_ref, l1b_ref,
             msg_ref, hout_ref):
  if first:
    z = h_ref[...]
    oh = (z == lax.broadcasted_iota(jnp.int32, (NB, HID), 1)).astype(jnp.float32)
    h = jnp.dot(oh, prevW_ref[...], preferred_element_type=jnp.float32)
  else:
    agg = a0_ref[...] + a1_ref[...]
    u = jnp.dot(agg, prevW_ref[...], preferred_element_type=jnp.float32) + prevb_ref[...]
    h = h_ref[...] + _silu(u)
  hout_ref[...] = h

  x = jnp.dot(h, l1w_ref[...], preferred_element_type=jnp.float32) + l1b_ref[...]

  d = d_ref[...]
  width = CUTOFF / (NG - 1)
  off = lax.broadcasted_iota(jnp.int32, (1, NGP), 1).astype(jnp.float32) * width
  t = (d - off) * (1.0 / width)
  rbf = jnp.exp(-0.5 * t * t)

  y = jnp.dot(rbf.astype(jnp.bfloat16), w1_ref[...].astype(jnp.bfloat16),
              preferred_element_type=jnp.float32) + b1_ref[...]
  y = _silu(y)
  w = jnp.dot(y.astype(jnp.bfloat16), w2_ref[...].astype(jnp.bfloat16),
              preferred_element_type=jnp.float32) + b2_ref[...]
  w = w * cw_ref[...]
  xrep = jnp.broadcast_to(x[:, None, :], (NB, K, HID)).reshape(NB * K, HID)
  msg_ref[...] = w * xrep


def _t1(first, h_or_z, a0, a1, d, cw, prevW, prevb, w1, b1, w2, b2, l1w, l1b):
  grid = N // NB
  full = lambda shape: pl.BlockSpec(shape, lambda i: tuple(0 for _ in shape))
  blk = lambda shape: pl.BlockSpec(shape, lambda i: (i,) + tuple(0 for _ in shape[1:]))
  hspec = blk((NB, 1)) if first else blk((NB, HID))
  return pl.pallas_call(
      functools.partial(_t1_body, first),
      grid=(grid,),
      in_specs=[
          hspec, blk((NB, HID)), blk((NB, HID)),
          blk((NB * K, 1)), blk((NB * K, 1)),
          full(prevW.shape), full(prevb.shape),
          full(w1.shape), full(b1.shape), full(w2.shape), full(b2.shape),
          full(l1w.shape), full(l1b.shape),
      ],
      out_specs=[
          blk((NB * K, HID)),
          blk((NB, HID)),
      ],
      out_shape=[
          jax.ShapeDtypeStruct((E, HID), jnp.float32),
          jax.ShapeDtypeStruct((N, HID), jnp.float32),
      ],
  )(h_or_z, a0, a1, d, cw, prevW, prevb, w1, b1, w2, b2, l1w, l1b)


NCHUNK = E // 128
CPS = NCHUNK // 2
CPT = CPS // 16
RPT = N // 16


def _sc_scatter(msg2d, nbr2d):
  mesh = plsc.VectorSubcoreMesh(core_axis_name="c", subcore_axis_name="s")

  @functools.partial(
      pl.kernel,
      out_type=jax.ShapeDtypeStruct((2, N, HID), jnp.float32),
      mesh=mesh,
      scratch_types=[
          pltpu.VMEM((CPT, 128), jnp.int32),
          pltpu.VMEM((128, HID), jnp.float32),
          pltpu.VMEM((128, HID), jnp.float32),
          pltpu.VMEM_SHARED((N, HID), jnp.float32),
          pltpu.SemaphoreType.DMA,
          pltpu.SemaphoreType.DMA,
      ],
  )
  def sc_kernel(msg_hbm, nbr_hbm, out_hbm, idx_v, buf0, buf1, agg_sh,
                sem0, sem1):
    c = lax.axis_index("c")
    s = lax.axis_index("s")

    def zrow(r, _):
      for q in range(HID // 16):
        buf0[r, pl.ds(q * 16, 16)] = jnp.zeros((16,), jnp.float32)
      return 0
    lax.fori_loop(0, 128, zrow, 0)

    def zcopy(q, _):
      pltpu.sync_copy(buf0, agg_sh.at[pl.ds(s * RPT + q * 128, 128)])
      return 0
    lax.fori_loop(0, RPT // 128, zcopy, 0)
    plsc.subcore_barrier()

    base = c * CPS + s * CPT
    pltpu.sync_copy(nbr_hbm.at[pl.ds(base, CPT)], idx_v)

    def gstart(ch, buf, sem):
      pltpu.async_copy(msg_hbm.at[pl.ds((base + ch) * 128, 128)], buf, sem)

    def gwait(ch, buf, sem):
      pltpu.make_async_copy(
          msg_hbm.at[pl.ds((base + ch) * 128, 128)], buf, sem).wait()

    gstart(0, buf0, sem0)

    def body(k2, _):
      ca = 2 * k2
      gwait(ca, buf0, sem0)
      gstart(ca + 1, buf1, sem1)
      pltpu.sync_copy(buf0, agg_sh.at[idx_v.at[ca]], add=True)
      gwait(ca + 1, buf1, sem1)

      @pl.when(k2 + 1 < CPT // 2)
      def _():
        gstart(ca + 2, buf0, sem0)

      pltpu.sync_copy(buf1, agg_sh.at[idx_v.at[ca + 1]], add=True)
      return 0

    lax.fori_loop(0, CPT // 2, body, 0)
    plsc.subcore_barrier()

    def wcopy(q, _):
      rows = pl.ds(s * RPT + q * 128, 128)
      pltpu.sync_copy(agg_sh.at[rows], out_hbm.at[c, rows])
      return 0
    lax.fori_loop(0, RPT // 128, wcopy, 0)

  return sc_kernel(msg2d, nbr2d)


def _tf_body(h_ref, a0_ref, a1_ref, batch_ref, l2w_ref, l2b_ref,
             o1w_ref, o1b_ref, o2w_ref, o2b_ref, out_ref):
  agg = a0_ref[...] + a1_ref[...]
  u = jnp.dot(agg, l2w_ref[...], preferred_element_type=jnp.float32) + l2b_ref[...]
  h = h_ref[...] + _silu(u)
  y = _silu(jnp.dot(h, o1w_ref[...], preferred_element_type=jnp.float32) + o1b_ref[...])
  o = jnp.sum(y * o2w_ref[...], axis=1, keepdims=True) + o2b_ref[...]
  oh = (batch_ref[...] == lax.broadcasted_iota(jnp.int32, (N, NGRAPH), 1))
  out_ref[...] = jnp.sum(jnp.where(oh, o, 0.0), axis=0, keepdims=True)


def _tf(h, a0, a1, batch2, l2w, l2b, o1w, o1b, o2w, o2b):
  full = lambda a: pl.BlockSpec(a.shape, lambda: tuple(0 for _ in a.shape))
  return pl.pallas_call(
      _tf_body,
      in_specs=[full(h), full(a0), full(a1), full(batch2), full(l2w),
                full(l2b), full(o1w), full(o1b), full(o2w), full(o2b)],
      out_specs=pl.BlockSpec((1, NGRAPH), lambda: (0, 0)),
      out_shape=jax.ShapeDtypeStruct((1, NGRAPH), jnp.float32),
  )(h, a0, a1, batch2, l2w, l2b, o1w, o1b, o2w, o2b)


def _pad_rows(w, rows):
  return jnp.concatenate([w, jnp.zeros((rows - w.shape[0],) + w.shape[1:], w.dtype)], 0)


def kernel(z, pos, batch, params):
  z = z.astype(jnp.int32)
  batch = batch.astype(jnp.int32)

  gids = jnp.arange(NGRAPH, dtype=jnp.int32)
  seg_lo = jnp.searchsorted(batch, gids, side="left").astype(jnp.int32)
  seg_hi = jnp.searchsorted(batch, gids, side="right").astype(jnp.int32)
  first_rows = batch[jnp.arange(N // RB, dtype=jnp.int32) * RB]
  last_rows = batch[jnp.arange(N // RB, dtype=jnp.int32) * RB + (RB - 1)]
  c0s = seg_lo[first_rows]
  c1s = seg_hi[last_rows]

  nbr, d, cw = _neighbor_search(pos, batch, c0s, c1s)
  nbr2d = nbr.reshape(NCHUNK, 128)
  d = d.reshape(E, 1)
  cw = cw.reshape(E, 1)

  emb_p = _pad_rows(params["emb"], HID)
  inter = params["inter"]

  h = z.reshape(N, 1)
  a0 = jnp.zeros((N, HID), jnp.float32)
  a1 = a0
  for li, p in enumerate(inter):
    if li == 0:
      prevW, prevb = emb_p, jnp.zeros((1, HID), jnp.float32)
    else:
      q = inter[li - 1]
      prevW, prevb = q["lin2"]["W"].T, q["lin2"]["b"].reshape(1, HID)
    w1 = _pad_rows(p["mlp1"]["W"].T, NGP)
    b1 = p["mlp1"]["b"].reshape(1, HID)
    w2 = p["mlp2"]["W"].T
    b2 = p["mlp2"]["b"].reshape(1, HID)
    l1w = p["lin1"]["W"].T
    l1b = p["lin1"]["b"].reshape(1, HID)
    msg, h = _t1(li == 0, h, a0, a1, d, cw, prevW, prevb, w1, b1, w2, b2,
                 l1w, l1b)
    aggs = _sc_scatter(msg, nbr2d)
    a0, a1 = aggs[0], aggs[1]

  q = inter[-1]
  l2w, l2b = q["lin2"]["W"].T, q["lin2"]["b"].reshape(1, HID)
  o1w = jnp.concatenate([params["out1"]["W"].T,
                         jnp.zeros((HID, HID // 2), jnp.float32)], axis=1)
  o1b = jnp.concatenate([params["out1"]["b"],
                         jnp.zeros((HID // 2,), jnp.float32)]).reshape(1, HID)
  o2w = jnp.concatenate([params["out2"]["W"][0],
                         jnp.zeros((HID // 2,), jnp.float32)]).reshape(1, HID)
  o2b = params["out2"]["b"].reshape(1, 1)
  res = _tf(h, a0, a1, batch.reshape(N, 1), l2w, l2b, o1w, o1b, o2w, o2b)
  return res.reshape(NGRAPH)


# speedup vs baseline: 1.2110x; 1.2110x over previous
import functools
import math

import jax
import jax.numpy as jnp
from jax import lax
from jax.experimental import pallas as pl
from jax.experimental.pallas import tpu as pltpu
from jax.experimental.pallas import tpu_sc as plsc

N = 8192
K = 32
HID = 128
NG = 50
NGP = 64
NGRAPH = 256
CUTOFF = 10.0
MAX_Z = 100

RB = 128
CH = 512
NCH = N // CH

NB = 128
E = N * K

INVALID_D = 20.0


def _k0_body(c0_ref, c1_ref, pos_r_ref, posc_ref, batch_r_ref, batch_c_ref,
             nbr_ref, d_ref, cw_ref, D_ref):
  b = pl.program_id(0)
  c0 = c0_ref[b]
  c1 = c1_ref[b]
  cb0 = c0 // CH
  cb1 = (c1 + CH - 1) // CH
  rbase = b * RB

  batch_r = batch_r_ref[...]
  row_ids = rbase + lax.broadcasted_iota(jnp.int32, (RB, CH), 0)
  lane_ids = lax.broadcasted_iota(jnp.int32, (RB, CH), 1)

  def fill(cb, _):
    acc = jnp.zeros((RB, CH), jnp.float32)
    for k in range(3):
      pr = pos_r_ref[:, k:k + 1]
      pc = posc_ref[k, cb]
      dk = pr - pc[None, :]
      acc = acc + dk * dk
    bc = batch_c_ref[cb]
    col_ids = cb * CH + lane_ids
    mask = (batch_r == bc[None, :]) & (col_ids != row_ids) & (acc < CUTOFF * CUTOFF)
    D_ref[cb] = jnp.where(mask, acc, jnp.inf)
    return 0

  lax.fori_loop(cb0, cb1, fill, 0)

  BIGI = jnp.int32(1 << 30)

  def emit(j, mv, mi):
    valid = mv < jnp.float32(1e30)
    idx = jnp.where(valid, mi, 0)
    d = jnp.where(valid, jnp.sqrt(jnp.maximum(mv, 0.0)), INVALID_D)
    cw = jnp.where(valid, 0.5 * (jnp.cos(d * (math.pi / CUTOFF)) + 1.0), 0.0)
    nbr_ref[:, j:j + 1] = idx
    d_ref[:, j:j + 1] = d
    cw_ref[:, j:j + 1] = cw

  nchunks = cb1 - cb0

  @pl.when(nchunks <= 2)
  def _static_path():
    cbB = jnp.minimum(cb0 + 1, NCH - 1)
    vA = D_ref[cb0]
    vB = jnp.where(nchunks >= 2, D_ref[cbB], jnp.inf)
    v = jnp.concatenate([vA, vB], axis=1)
    cid2 = jnp.concatenate([cb0 * CH + lane_ids, cbB * CH + lane_ids], axis=1)
    for j in range(K):
      vmin = jnp.min(v, axis=1, keepdims=True)
      imin = jnp.min(jnp.where(v == vmin, cid2, BIGI), axis=1, keepdims=True)
      emit(j, vmin, imin)
      v = jnp.where((v == vmin) & (cid2 == imin), jnp.inf, v)

  @pl.when(nchunks > 2)
  def _dynamic_path():
    last_v = jnp.full((RB, 1), -jnp.inf, jnp.float32)
    last_i = jnp.full((RB, 1), -1, jnp.int32)
    for j in range(K):
      def scan_chunk(cb, carry):
        mv, mi = carry
        chunk = D_ref[cb]
        col_ids = cb * CH + lane_ids
        ok = (chunk > last_v) | ((chunk == last_v) & (col_ids > last_i))
        effv = jnp.where(ok, chunk, jnp.inf)
        effi = jnp.where(ok, col_ids, BIGI)
        vmin = jnp.min(effv, axis=1, keepdims=True)
        imin = jnp.min(jnp.where(effv == vmin, effi, BIGI), axis=1, keepdims=True)
        better = (vmin < mv) | ((vmin == mv) & (imin < mi))
        return (jnp.where(better, vmin, mv), jnp.where(better, imin, mi))

      mv0 = jnp.full((RB, 1), jnp.inf, jnp.float32)
      mi0 = jnp.full((RB, 1), BIGI, jnp.int32)
      mv, mi = lax.fori_loop(cb0, cb1, scan_chunk, (mv0, mi0))
      last_v, last_i = mv, mi
      emit(j, mv, mi)


def _neighbor_search(pos, batch, c0s, c1s):
  posc = pos.T.reshape(3, NCH, CH)
  batch_c = batch.reshape(NCH, CH)
  batch_r = batch.reshape(N, 1)
  grid = N // RB
  return pl.pallas_call(
      _k0_body,
      grid=(grid,),
      in_specs=[
          pl.BlockSpec(memory_space=pltpu.SMEM),
          pl.BlockSpec(memory_space=pltpu.SMEM),
          pl.BlockSpec((RB, 3), lambda b: (b, 0)),
          pl.BlockSpec((3, NCH, CH), lambda b: (0, 0, 0)),
          pl.BlockSpec((RB, 1), lambda b: (b, 0)),
          pl.BlockSpec((NCH, CH), lambda b: (0, 0)),
      ],
      out_specs=[
          pl.BlockSpec((RB, K), lambda b: (b, 0)),
          pl.BlockSpec((RB, K), lambda b: (b, 0)),
          pl.BlockSpec((RB, K), lambda b: (b, 0)),
      ],
      out_shape=[
          jax.ShapeDtypeStruct((N, K), jnp.int32),
          jax.ShapeDtypeStruct((N, K), jnp.float32),
          jax.ShapeDtypeStruct((N, K), jnp.float32),
      ],
      scratch_shapes=[pltpu.VMEM((NCH, RB, CH), jnp.float32)],
  )(c0s, c1s, pos, posc, batch_r, batch_c)


def _silu(x):
  return x * jax.nn.sigmoid(x)


def _t1_body(first, h_ref, a0_ref, a1_ref, d_ref, cw_ref, prevW_ref, prevb_ref,
             w1_ref, b1_ref, w2_ref, b2_ref, l1w_ref, l1b_ref,
             msg_ref, hout_ref):
  if first:
    z = h_ref[...]
    oh = (z == lax.broadcasted_iota(jnp.int32, (NB, HID), 1)).astype(jnp.float32)
    h = jnp.dot(oh, prevW_ref[...], preferred_element_type=jnp.float32)
  else:
    agg = a0_ref[...] + a1_ref[...]
    u = jnp.dot(agg, prevW_ref[...], preferred_element_type=jnp.float32) + prevb_ref[...]
    h = h_ref[...] + _silu(u)
  hout_ref[...] = h

  x = jnp.dot(h, l1w_ref[...], preferred_element_type=jnp.float32) + l1b_ref[...]

  d = d_ref[...]
  width = CUTOFF / (NG - 1)
  off = lax.broadcasted_iota(jnp.int32, (1, NGP), 1).astype(jnp.float32) * width
  t = (d - off) * (1.0 / width)
  rbf = jnp.exp(-0.5 * t * t)

  y = jnp.dot(rbf.astype(jnp.bfloat16), w1_ref[...].astype(jnp.bfloat16),
              preferred_element_type=jnp.float32) + b1_ref[...]
  y = _silu(y)
  w = jnp.dot(y.astype(jnp.bfloat16), w2_ref[...].astype(jnp.bfloat16),
              preferred_element_type=jnp.float32) + b2_ref[...]
  w = w * cw_ref[...]
  xrep = jnp.broadcast_to(x[:, None, :], (NB, K, HID)).reshape(NB * K, HID)
  msg_ref[...] = w * xrep


def _t1(first, h_or_z, a0, a1, d, cw, prevW, prevb, w1, b1, w2, b2, l1w, l1b):
  grid = N // NB
  full = lambda shape: pl.BlockSpec(shape, lambda i: tuple(0 for _ in shape))
  blk = lambda shape: pl.BlockSpec(shape, lambda i: (i,) + tuple(0 for _ in shape[1:]))
  hspec = blk((NB, 1)) if first else blk((NB, HID))
  return pl.pallas_call(
      functools.partial(_t1_body, first),
      grid=(grid,),
      in_specs=[
          hspec, blk((NB, HID)), blk((NB, HID)),
          blk((NB * K, 1)), blk((NB * K, 1)),
          full(prevW.shape), full(prevb.shape),
          full(w1.shape), full(b1.shape), full(w2.shape), full(b2.shape),
          full(l1w.shape), full(l1b.shape),
      ],
      out_specs=[
          blk((NB * K, HID)),
          blk((NB, HID)),
      ],
      out_shape=[
          jax.ShapeDtypeStruct((E, HID), jnp.float32),
          jax.ShapeDtypeStruct((N, HID), jnp.float32),
      ],
  )(h_or_z, a0, a1, d, cw, prevW, prevb, w1, b1, w2, b2, l1w, l1b)


NCHUNK = E // 128
CPS = NCHUNK // 2
CPT = CPS // 16
RPT = N // 16


def _sc_scatter(msg2d, nbr2d):
  mesh = plsc.VectorSubcoreMesh(core_axis_name="c", subcore_axis_name="s")

  @functools.partial(
      pl.kernel,
      out_type=jax.ShapeDtypeStruct((2, N, HID), jnp.float32),
      mesh=mesh,
      scratch_types=[
          pltpu.VMEM((CPT, 128), jnp.int32),
          pltpu.VMEM((128, HID), jnp.float32),
          pltpu.VMEM((128, HID), jnp.float32),
          pltpu.VMEM_SHARED((N, HID), jnp.float32),
          pltpu.SemaphoreType.DMA,
          pltpu.SemaphoreType.DMA,
      ],
  )
  def sc_kernel(msg_hbm, nbr_hbm, out_hbm, idx_v, buf0, buf1, agg_sh,
                sem0, sem1):
    c = lax.axis_index("c")
    s = lax.axis_index("s")

    def zrow(r, _):
      for q in range(HID // 16):
        buf0[r, pl.ds(q * 16, 16)] = jnp.zeros((16,), jnp.float32)
      return 0
    lax.fori_loop(0, 128, zrow, 0)

    def zcopy(q, _):
      pltpu.sync_copy(buf0, agg_sh.at[pl.ds(s * RPT + q * 128, 128)])
      return 0
    lax.fori_loop(0, RPT // 128, zcopy, 0)
    plsc.subcore_barrier()

    base = c * CPS + s * CPT
    pltpu.sync_copy(nbr_hbm.at[pl.ds(base, CPT)], idx_v)

    def gstart(ch, buf, sem):
      pltpu.async_copy(msg_hbm.at[pl.ds((base + ch) * 128, 128)], buf, sem)

    def gwait(ch, buf, sem):
      pltpu.make_async_copy(
          msg_hbm.at[pl.ds((base + ch) * 128, 128)], buf, sem).wait()

    gstart(0, buf0, sem0)

    def body(k2, _):
      ca = 2 * k2
      gwait(ca, buf0, sem0)
      gstart(ca + 1, buf1, sem1)
      pltpu.sync_copy(buf0, agg_sh.at[idx_v.at[ca]], add=True)
      gwait(ca + 1, buf1, sem1)

      @pl.when(k2 + 1 < CPT // 2)
      def _():
        gstart(ca + 2, buf0, sem0)

      pltpu.sync_copy(buf1, agg_sh.at[idx_v.at[ca + 1]], add=True)
      return 0

    lax.fori_loop(0, CPT // 2, body, 0)
    plsc.subcore_barrier()

    def wcopy(q, _):
      rows = pl.ds(s * RPT + q * 128, 128)
      pltpu.sync_copy(agg_sh.at[rows], out_hbm.at[c, rows])
      return 0
    lax.fori_loop(0, RPT // 128, wcopy, 0)

  return sc_kernel(msg2d, nbr2d)


def _tf_body(h_ref, a0_ref, a1_ref, batch_ref, l2w_ref, l2b_ref,
             o1w_ref, o1b_ref, o2w_ref, o2b_ref, out_ref):
  agg = a0_ref[...] + a1_ref[...]
  u = jnp.dot(agg, l2w_ref[...], preferred_element_type=jnp.float32) + l2b_ref[...]
  h = h_ref[...] + _silu(u)
  y = _silu(jnp.dot(h, o1w_ref[...], preferred_element_type=jnp.float32) + o1b_ref[...])
  o = jnp.sum(y * o2w_ref[...], axis=1, keepdims=True) + o2b_ref[...]
  oh = (batch_ref[...] == lax.broadcasted_iota(jnp.int32, (N, NGRAPH), 1))
  out_ref[...] = jnp.sum(jnp.where(oh, o, 0.0), axis=0, keepdims=True)


def _tf(h, a0, a1, batch2, l2w, l2b, o1w, o1b, o2w, o2b):
  full = lambda a: pl.BlockSpec(a.shape, lambda: tuple(0 for _ in a.shape))
  return pl.pallas_call(
      _tf_body,
      in_specs=[full(h), full(a0), full(a1), full(batch2), full(l2w),
                full(l2b), full(o1w), full(o1b), full(o2w), full(o2b)],
      out_specs=pl.BlockSpec((1, NGRAPH), lambda: (0, 0)),
      out_shape=jax.ShapeDtypeStruct((1, NGRAPH), jnp.float32),
  )(h, a0, a1, batch2, l2w, l2b, o1w, o1b, o2w, o2b)


def _pad_rows(w, rows):
  return jnp.concatenate([w, jnp.zeros((rows - w.shape[0],) + w.shape[1:], w.dtype)], 0)


def kernel(z, pos, batch, params):
  z = z.astype(jnp.int32)
  batch = batch.astype(jnp.int32)

  gids = jnp.arange(NGRAPH, dtype=jnp.int32)
  seg_lo = jnp.searchsorted(batch, gids, side="left").astype(jnp.int32)
  seg_hi = jnp.searchsorted(batch, gids, side="right").astype(jnp.int32)
  first_rows = batch[jnp.arange(N // RB, dtype=jnp.int32) * RB]
  last_rows = batch[jnp.arange(N // RB, dtype=jnp.int32) * RB + (RB - 1)]
  c0s = seg_lo[first_rows]
  c1s = seg_hi[last_rows]

  nbr, d, cw = _neighbor_search(pos, batch, c0s, c1s)
  nbr2d = nbr.reshape(NCHUNK, 128)
  d = d.reshape(E, 1)
  cw = cw.reshape(E, 1)

  emb_p = _pad_rows(params["emb"], HID)
  inter = params["inter"]

  h = z.reshape(N, 1)
  a0 = jnp.zeros((N, HID), jnp.float32)
  a1 = a0
  for li, p in enumerate(inter):
    if li == 0:
      prevW, prevb = emb_p, jnp.zeros((1, HID), jnp.float32)
    else:
      q = inter[li - 1]
      prevW, prevb = q["lin2"]["W"].T, q["lin2"]["b"].reshape(1, HID)
    w1 = _pad_rows(p["mlp1"]["W"].T, NGP)
    b1 = p["mlp1"]["b"].reshape(1, HID)
    w2 = p["mlp2"]["W"].T
    b2 = p["mlp2"]["b"].reshape(1, HID)
    l1w = p["lin1"]["W"].T
    l1b = p["lin1"]["b"].reshape(1, HID)
    msg, h = _t1(li == 0, h, a0, a1, d, cw, prevW, prevb, w1, b1, w2, b2,
                 l1w, l1b)
    aggs = _sc_scatter(msg, nbr2d)
    a0, a1 = aggs[0], aggs[1]

  q = inter[-1]
  l2w, l2b = q["lin2"]["W"].T, q["lin2"]["b"].reshape(1, HID)
  o1w = jnp.concatenate([params["out1"]["W"].T,
                         jnp.zeros((HID, HID // 2), jnp.float32)], axis=1)
  o1b = jnp.concatenate([params["out1"]["b"],
                         jnp.zeros((HID // 2,), jnp.float32)]).reshape(1, HID)
  o2w = jnp.concatenate([params["out2"]["W"][0],
                         jnp.zeros((HID // 2,), jnp.float32)]).reshape(1, HID)
  o2b = params["out2"]["b"].reshape(1, 1)
  res = _tf(h, a0, a1, batch.reshape(N, 1), l2w, l2b, o1w, o1b, o2w, o2b)
  return res.reshape(NGRAPH)


# speedup vs baseline: 1.2121x; 1.0009x over previous
import functools
import math

import jax
import jax.numpy as jnp
from jax import lax
from jax.experimental import pallas as pl
from jax.experimental.pallas import tpu as pltpu
from jax.experimental.pallas import tpu_sc as plsc

N = 8192
K = 32
HID = 128
NG = 50
NGP = 64
NGRAPH = 256
CUTOFF = 10.0
MAX_Z = 100

RB = 128
CH = 512
NCH = N // CH

NB = 128
E = N * K

INVALID_D = 20.0


def _k0_body(c0_ref, c1_ref, pos_r_ref, posc_ref, batch_r_ref, batch_c_ref,
             nbr_ref, d_ref, cw_ref, D_ref):
  b = pl.program_id(0)
  c0 = c0_ref[b]
  c1 = c1_ref[b]
  cb0 = c0 // CH
  cb1 = (c1 + CH - 1) // CH
  rbase = b * RB

  batch_r = batch_r_ref[...]
  row_ids = rbase + lax.broadcasted_iota(jnp.int32, (RB, CH), 0)
  lane_ids = lax.broadcasted_iota(jnp.int32, (RB, CH), 1)

  def fill(cb, _):
    acc = jnp.zeros((RB, CH), jnp.float32)
    for k in range(3):
      pr = pos_r_ref[:, k:k + 1]
      pc = posc_ref[k, cb]
      dk = pr - pc[None, :]
      acc = acc + dk * dk
    bc = batch_c_ref[cb]
    col_ids = cb * CH + lane_ids
    mask = (batch_r == bc[None, :]) & (col_ids != row_ids) & (acc < CUTOFF * CUTOFF)
    D_ref[cb] = jnp.where(mask, acc, jnp.inf)
    return 0

  lax.fori_loop(cb0, cb1, fill, 0)

  BIGI = jnp.int32(1 << 30)

  def emit(j, mv, mi):
    valid = mv < jnp.float32(1e30)
    idx = jnp.where(valid, mi, 0)
    d = jnp.where(valid, jnp.sqrt(jnp.maximum(mv, 0.0)), INVALID_D)
    cw = jnp.where(valid, 0.5 * (jnp.cos(d * (math.pi / CUTOFF)) + 1.0), 0.0)
    nbr_ref[:, j:j + 1] = idx
    d_ref[:, j:j + 1] = d
    cw_ref[:, j:j + 1] = cw

  nchunks = cb1 - cb0

  @pl.when(nchunks <= 2)
  def _static_path():
    cbB = jnp.minimum(cb0 + 1, NCH - 1)
    vA = D_ref[cb0]
    vB = jnp.where(nchunks >= 2, D_ref[cbB], jnp.inf)
    v = jnp.concatenate([vA, vB], axis=1)
    cid2 = jnp.concatenate([cb0 * CH + lane_ids, cbB * CH + lane_ids], axis=1)
    for j in range(K):
      vmin = jnp.min(v, axis=1, keepdims=True)
      imin = jnp.min(jnp.where(v == vmin, cid2, BIGI), axis=1, keepdims=True)
      emit(j, vmin, imin)
      v = jnp.where((v == vmin) & (cid2 == imin), jnp.inf, v)

  @pl.when(nchunks > 2)
  def _dynamic_path():
    last_v = jnp.full((RB, 1), -jnp.inf, jnp.float32)
    last_i = jnp.full((RB, 1), -1, jnp.int32)
    for j in range(K):
      def scan_chunk(cb, carry):
        mv, mi = carry
        chunk = D_ref[cb]
        col_ids = cb * CH + lane_ids
        ok = (chunk > last_v) | ((chunk == last_v) & (col_ids > last_i))
        effv = jnp.where(ok, chunk, jnp.inf)
        effi = jnp.where(ok, col_ids, BIGI)
        vmin = jnp.min(effv, axis=1, keepdims=True)
        imin = jnp.min(jnp.where(effv == vmin, effi, BIGI), axis=1, keepdims=True)
        better = (vmin < mv) | ((vmin == mv) & (imin < mi))
        return (jnp.where(better, vmin, mv), jnp.where(better, imin, mi))

      mv0 = jnp.full((RB, 1), jnp.inf, jnp.float32)
      mi0 = jnp.full((RB, 1), BIGI, jnp.int32)
      mv, mi = lax.fori_loop(cb0, cb1, scan_chunk, (mv0, mi0))
      last_v, last_i = mv, mi
      emit(j, mv, mi)


def _neighbor_search(pos, batch, c0s, c1s):
  posc = pos.T.reshape(3, NCH, CH)
  batch_c = batch.reshape(NCH, CH)
  batch_r = batch.reshape(N, 1)
  grid = N // RB
  return pl.pallas_call(
      _k0_body,
      grid=(grid,),
      in_specs=[
          pl.BlockSpec(memory_space=pltpu.SMEM),
          pl.BlockSpec(memory_space=pltpu.SMEM),
          pl.BlockSpec((RB, 3), lambda b: (b, 0)),
          pl.BlockSpec((3, NCH, CH), lambda b: (0, 0, 0)),
          pl.BlockSpec((RB, 1), lambda b: (b, 0)),
          pl.BlockSpec((NCH, CH), lambda b: (0, 0)),
      ],
      out_specs=[
          pl.BlockSpec((RB, K), lambda b: (b, 0)),
          pl.BlockSpec((RB, K), lambda b: (b, 0)),
          pl.BlockSpec((RB, K), lambda b: (b, 0)),
      ],
      out_shape=[
          jax.ShapeDtypeStruct((N, K), jnp.int32),
          jax.ShapeDtypeStruct((N, K), jnp.float32),
          jax.ShapeDtypeStruct((N, K), jnp.float32),
      ],
      scratch_shapes=[pltpu.VMEM((NCH, RB, CH), jnp.float32)],
  )(c0s, c1s, pos, posc, batch_r, batch_c)


def _silu(x):
  return x * jax.nn.sigmoid(x)


def _t1_body(first, h_ref, a0_ref, a1_ref, d_ref, cw_ref, prevW_ref, prevb_ref,
             w1_ref, b1_ref, w2_ref, b2_ref, l1w_ref, l1b_ref,
             msg_ref, hout_ref):
  if first:
    z = h_ref[...]
    oh = (z == lax.broadcasted_iota(jnp.int32, (NB, HID), 1)).astype(jnp.float32)
    h = jnp.dot(oh, prevW_ref[...], preferred_element_type=jnp.float32)
  else:
    agg = a0_ref[...] + a1_ref[...]
    u = jnp.dot(agg, prevW_ref[...], preferred_element_type=jnp.float32) + prevb_ref[...]
    h = h_ref[...] + _silu(u)
  hout_ref[...] = h

  x = jnp.dot(h, l1w_ref[...], preferred_element_type=jnp.float32) + l1b_ref[...]

  d = d_ref[...]
  width = CUTOFF / (NG - 1)
  off = lax.broadcasted_iota(jnp.int32, (1, NGP), 1).astype(jnp.float32) * width
  t = (d - off) * (1.0 / width)
  rbf = jnp.exp(-0.5 * t * t)

  y = jnp.dot(rbf.astype(jnp.bfloat16), w1_ref[...].astype(jnp.bfloat16),
              preferred_element_type=jnp.float32) + b1_ref[...]
  y = _silu(y)
  w = jnp.dot(y.astype(jnp.bfloat16), w2_ref[...].astype(jnp.bfloat16),
              preferred_element_type=jnp.float32) + b2_ref[...]
  w = w * cw_ref[...]
  xrep = jnp.broadcast_to(x[:, None, :], (NB, K, HID)).reshape(NB * K, HID)
  msg_ref[...] = w * xrep


def _t1(first, h_or_z, a0, a1, d, cw, prevW, prevb, w1, b1, w2, b2, l1w, l1b):
  grid = N // NB
  full = lambda shape: pl.BlockSpec(shape, lambda i: tuple(0 for _ in shape))
  blk = lambda shape: pl.BlockSpec(shape, lambda i: (i,) + tuple(0 for _ in shape[1:]))
  hspec = blk((NB, 1)) if first else blk((NB, HID))
  aspec = full((1, HID)) if first else blk((NB, HID))
  return pl.pallas_call(
      functools.partial(_t1_body, first),
      grid=(grid,),
      in_specs=[
          hspec, aspec, aspec,
          blk((NB * K, 1)), blk((NB * K, 1)),
          full(prevW.shape), full(prevb.shape),
          full(w1.shape), full(b1.shape), full(w2.shape), full(b2.shape),
          full(l1w.shape), full(l1b.shape),
      ],
      out_specs=[
          blk((NB * K, HID)),
          blk((NB, HID)),
      ],
      out_shape=[
          jax.ShapeDtypeStruct((E, HID), jnp.float32),
          jax.ShapeDtypeStruct((N, HID), jnp.float32),
      ],
  )(h_or_z, a0, a1, d, cw, prevW, prevb, w1, b1, w2, b2, l1w, l1b)


NCHUNK = E // 128
CPS = NCHUNK // 2
CPT = CPS // 16
RPT = N // 16


def _sc_scatter(msg2d, nbr2d):
  mesh = plsc.VectorSubcoreMesh(core_axis_name="c", subcore_axis_name="s")

  @functools.partial(
      pl.kernel,
      out_type=jax.ShapeDtypeStruct((2, N, HID), jnp.float32),
      mesh=mesh,
      scratch_types=[
          pltpu.VMEM((CPT, 128), jnp.int32),
          pltpu.VMEM((128, HID), jnp.float32),
          pltpu.VMEM((128, HID), jnp.float32),
          pltpu.VMEM_SHARED((N, HID), jnp.float32),
          pltpu.SemaphoreType.DMA,
          pltpu.SemaphoreType.DMA,
      ],
  )
  def sc_kernel(msg_hbm, nbr_hbm, out_hbm, idx_v, buf0, buf1, agg_sh,
                sem0, sem1):
    c = lax.axis_index("c")
    s = lax.axis_index("s")

    def zrow(r, _):
      for q in range(HID // 16):
        buf0[r, pl.ds(q * 16, 16)] = jnp.zeros((16,), jnp.float32)
      return 0
    lax.fori_loop(0, 128, zrow, 0)

    def zcopy(q, _):
      pltpu.sync_copy(buf0.at[pl.ds(0, 128)],
                      agg_sh.at[pl.ds(s * RPT + q * 128, 128)])
      return 0
    lax.fori_loop(0, RPT // 128, zcopy, 0)
    plsc.subcore_barrier()

    base = c * CPS + s * CPT

    def gstart(ch, buf, sem):
      pltpu.async_copy(msg_hbm.at[pl.ds((base + ch) * 128, 128)], buf, sem)

    def gwait(ch, buf, sem):
      pltpu.make_async_copy(
          msg_hbm.at[pl.ds((base + ch) * 128, 128)], buf, sem).wait()

    pltpu.sync_copy(nbr_hbm.at[pl.ds(base, CPT)], idx_v)
    gstart(0, buf0, sem0)

    def body(k2, _):
      ca = 2 * k2
      gwait(ca, buf0, sem0)
      gstart(ca + 1, buf1, sem1)
      pltpu.sync_copy(buf0, agg_sh.at[idx_v.at[ca]], add=True)
      gwait(ca + 1, buf1, sem1)

      @pl.when(k2 + 1 < CPT // 2)
      def _():
        gstart(ca + 2, buf0, sem0)

      pltpu.sync_copy(buf1, agg_sh.at[idx_v.at[ca + 1]], add=True)
      return 0

    lax.fori_loop(0, CPT // 2, body, 0)
    plsc.subcore_barrier()

    def wcopy(q, _):
      rows = pl.ds(s * RPT + q * 128, 128)
      pltpu.sync_copy(agg_sh.at[rows], out_hbm.at[c, rows])
      return 0
    lax.fori_loop(0, RPT // 128, wcopy, 0)

  return sc_kernel(msg2d, nbr2d)


def _tf_body(h_ref, a0_ref, a1_ref, batch_ref, l2w_ref, l2b_ref,
             o1w_ref, o1b_ref, o2w_ref, o2b_ref, out_ref):
  agg = a0_ref[...] + a1_ref[...]
  u = jnp.dot(agg, l2w_ref[...], preferred_element_type=jnp.float32) + l2b_ref[...]
  h = h_ref[...] + _silu(u)
  y = _silu(jnp.dot(h, o1w_ref[...], preferred_element_type=jnp.float32) + o1b_ref[...])
  o = jnp.sum(y * o2w_ref[...], axis=1, keepdims=True) + o2b_ref[...]
  oh = (batch_ref[...] == lax.broadcasted_iota(jnp.int32, (N, NGRAPH), 1))
  out_ref[...] = jnp.sum(jnp.where(oh, o, 0.0), axis=0, keepdims=True)


def _tf(h, a0, a1, batch2, l2w, l2b, o1w, o1b, o2w, o2b):
  full = lambda a: pl.BlockSpec(a.shape, lambda: tuple(0 for _ in a.shape))
  return pl.pallas_call(
      _tf_body,
      in_specs=[full(h), full(a0), full(a1), full(batch2), full(l2w),
                full(l2b), full(o1w), full(o1b), full(o2w), full(o2b)],
      out_specs=pl.BlockSpec((1, NGRAPH), lambda: (0, 0)),
      out_shape=jax.ShapeDtypeStruct((1, NGRAPH), jnp.float32),
  )(h, a0, a1, batch2, l2w, l2b, o1w, o1b, o2w, o2b)


def _pad_rows(w, rows):
  return jnp.concatenate([w, jnp.zeros((rows - w.shape[0],) + w.shape[1:], w.dtype)], 0)


def kernel(z, pos, batch, params):
  z = z.astype(jnp.int32)
  batch = batch.astype(jnp.int32)

  gids = jnp.arange(NGRAPH, dtype=jnp.int32)
  seg_lo = jnp.searchsorted(batch, gids, side="left").astype(jnp.int32)
  seg_hi = jnp.searchsorted(batch, gids, side="right").astype(jnp.int32)
  first_rows = batch[jnp.arange(N // RB, dtype=jnp.int32) * RB]
  last_rows = batch[jnp.arange(N // RB, dtype=jnp.int32) * RB + (RB - 1)]
  c0s = seg_lo[first_rows]
  c1s = seg_hi[last_rows]

  nbr, d, cw = _neighbor_search(pos, batch, c0s, c1s)
  nbr2d = nbr.reshape(NCHUNK, 128)
  d = d.reshape(E, 1)
  cw = cw.reshape(E, 1)

  emb_p = _pad_rows(params["emb"], HID)
  inter = params["inter"]

  h = z.reshape(N, 1)
  a0 = jnp.zeros((1, HID), jnp.float32)
  a1 = a0
  for li, p in enumerate(inter):
    if li == 0:
      prevW, prevb = emb_p, jnp.zeros((1, HID), jnp.float32)
    else:
      q = inter[li - 1]
      prevW, prevb = q["lin2"]["W"].T, q["lin2"]["b"].reshape(1, HID)
    w1 = _pad_rows(p["mlp1"]["W"].T, NGP)
    b1 = p["mlp1"]["b"].reshape(1, HID)
    w2 = p["mlp2"]["W"].T
    b2 = p["mlp2"]["b"].reshape(1, HID)
    l1w = p["lin1"]["W"].T
    l1b = p["lin1"]["b"].reshape(1, HID)
    msg, h = _t1(li == 0, h, a0, a1, d, cw, prevW, prevb, w1, b1, w2, b2,
                 l1w, l1b)
    aggs = _sc_scatter(msg, nbr2d)
    a0, a1 = aggs[0], aggs[1]

  q = inter[-1]
  l2w, l2b = q["lin2"]["W"].T, q["lin2"]["b"].reshape(1, HID)
  o1w = jnp.concatenate([params["out1"]["W"].T,
                         jnp.zeros((HID, HID // 2), jnp.float32)], axis=1)
  o1b = jnp.concatenate([params["out1"]["b"],
                         jnp.zeros((HID // 2,), jnp.float32)]).reshape(1, HID)
  o2w = jnp.concatenate([params["out2"]["W"][0],
                         jnp.zeros((HID // 2,), jnp.float32)]).reshape(1, HID)
  o2b = params["out2"]["b"].reshape(1, 1)
  res = _tf(h, a0, a1, batch.reshape(N, 1), l2w, l2b, o1w, o1b, o2w, o2b)
  return res.reshape(NGRAPH)


# speedup vs baseline: 1.5269x; 1.2597x over previous
import functools
import math

import jax
import jax.numpy as jnp
from jax import lax
from jax.experimental import pallas as pl
from jax.experimental.pallas import tpu as pltpu
from jax.experimental.pallas import tpu_sc as plsc

N = 8192
K = 32
HID = 128
NG = 50
NGP = 64
NGRAPH = 256
CUTOFF = 10.0
MAX_Z = 100

RB = 128
CH = 512
NCH = N // CH

NB = 128
E = N * K

INVALID_D = 20.0


def _k0_body(c0_ref, c1_ref, pos_r_ref, posc_ref, batch_r_ref, batch_c_ref,
             nbr_ref, d_ref, cw_ref, D_ref):
  b = pl.program_id(0)
  c0 = c0_ref[b]
  c1 = c1_ref[b]
  cb0 = c0 // CH
  cb1 = (c1 + CH - 1) // CH
  rbase = b * RB

  batch_r = batch_r_ref[...]
  row_ids = rbase + lax.broadcasted_iota(jnp.int32, (RB, CH), 0)
  lane_ids = lax.broadcasted_iota(jnp.int32, (RB, CH), 1)

  def fill(cb, _):
    acc = jnp.zeros((RB, CH), jnp.float32)
    for k in range(3):
      pr = pos_r_ref[:, k:k + 1]
      pc = posc_ref[k, cb]
      dk = pr - pc[None, :]
      acc = acc + dk * dk
    bc = batch_c_ref[cb]
    col_ids = cb * CH + lane_ids
    mask = (batch_r == bc[None, :]) & (col_ids != row_ids) & (acc < CUTOFF * CUTOFF)
    D_ref[cb] = jnp.where(mask, acc, jnp.inf)
    return 0

  lax.fori_loop(cb0, cb1, fill, 0)

  BIGI = jnp.int32(1 << 30)

  def emit(j, mv, mi):
    valid = mv < jnp.float32(1e30)
    idx = jnp.where(valid, mi, 0)
    d = jnp.where(valid, jnp.sqrt(jnp.maximum(mv, 0.0)), INVALID_D)
    cw = jnp.where(valid, 0.5 * (jnp.cos(d * (math.pi / CUTOFF)) + 1.0), 0.0)
    nbr_ref[:, j:j + 1] = idx
    d_ref[:, j:j + 1] = d
    cw_ref[:, j:j + 1] = cw

  nchunks = cb1 - cb0

  @pl.when(nchunks <= 2)
  def _static_path():
    cbB = jnp.minimum(cb0 + 1, NCH - 1)
    vA = D_ref[cb0]
    vB = jnp.where(nchunks >= 2, D_ref[cbB], jnp.inf)
    v = jnp.concatenate([vA, vB], axis=1)
    cid2 = jnp.concatenate([cb0 * CH + lane_ids, cbB * CH + lane_ids], axis=1)
    for j in range(K):
      vmin = jnp.min(v, axis=1, keepdims=True)
      imin = jnp.min(jnp.where(v == vmin, cid2, BIGI), axis=1, keepdims=True)
      emit(j, vmin, imin)
      v = jnp.where((v == vmin) & (cid2 == imin), jnp.inf, v)

  @pl.when(nchunks > 2)
  def _dynamic_path():
    last_v = jnp.full((RB, 1), -jnp.inf, jnp.float32)
    last_i = jnp.full((RB, 1), -1, jnp.int32)
    for j in range(K):
      def scan_chunk(cb, carry):
        mv, mi = carry
        chunk = D_ref[cb]
        col_ids = cb * CH + lane_ids
        ok = (chunk > last_v) | ((chunk == last_v) & (col_ids > last_i))
        effv = jnp.where(ok, chunk, jnp.inf)
        effi = jnp.where(ok, col_ids, BIGI)
        vmin = jnp.min(effv, axis=1, keepdims=True)
        imin = jnp.min(jnp.where(effv == vmin, effi, BIGI), axis=1, keepdims=True)
        better = (vmin < mv) | ((vmin == mv) & (imin < mi))
        return (jnp.where(better, vmin, mv), jnp.where(better, imin, mi))

      mv0 = jnp.full((RB, 1), jnp.inf, jnp.float32)
      mi0 = jnp.full((RB, 1), BIGI, jnp.int32)
      mv, mi = lax.fori_loop(cb0, cb1, scan_chunk, (mv0, mi0))
      last_v, last_i = mv, mi
      emit(j, mv, mi)


def _neighbor_search(pos, batch, c0s, c1s):
  posc = pos.T.reshape(3, NCH, CH)
  batch_c = batch.reshape(NCH, CH)
  batch_r = batch.reshape(N, 1)
  grid = N // RB
  return pl.pallas_call(
      _k0_body,
      grid=(grid,),
      in_specs=[
          pl.BlockSpec(memory_space=pltpu.SMEM),
          pl.BlockSpec(memory_space=pltpu.SMEM),
          pl.BlockSpec((RB, 3), lambda b: (b, 0)),
          pl.BlockSpec((3, NCH, CH), lambda b: (0, 0, 0)),
          pl.BlockSpec((RB, 1), lambda b: (b, 0)),
          pl.BlockSpec((NCH, CH), lambda b: (0, 0)),
      ],
      out_specs=[
          pl.BlockSpec((RB, K), lambda b: (b, 0)),
          pl.BlockSpec((RB, K), lambda b: (b, 0)),
          pl.BlockSpec((RB, K), lambda b: (b, 0)),
      ],
      out_shape=[
          jax.ShapeDtypeStruct((N, K), jnp.int32),
          jax.ShapeDtypeStruct((N, K), jnp.float32),
          jax.ShapeDtypeStruct((N, K), jnp.float32),
      ],
      scratch_shapes=[pltpu.VMEM((NCH, RB, CH), jnp.float32)],
  )(c0s, c1s, pos, posc, batch_r, batch_c)


def _silu(x):
  return x * jax.nn.sigmoid(x)


def _t1_body(first, h_ref, a0_ref, a1_ref, d_ref, cw_ref, prevW_ref, prevb_ref,
             w1_ref, b1_ref, w2_ref, b2_ref, l1w_ref, l1b_ref,
             msg_ref, hout_ref):
  if first:
    z = h_ref[...]
    oh = (z == lax.broadcasted_iota(jnp.int32, (NB, HID), 1)).astype(jnp.float32)
    h = jnp.dot(oh, prevW_ref[...], preferred_element_type=jnp.float32)
  else:
    agg = a0_ref[...] + a1_ref[...]
    u = jnp.dot(agg, prevW_ref[...], preferred_element_type=jnp.float32) + prevb_ref[...]
    h = h_ref[...] + _silu(u)
  hout_ref[...] = h

  x = jnp.dot(h, l1w_ref[...], preferred_element_type=jnp.float32) + l1b_ref[...]

  d = d_ref[...][:, :, None]
  width = CUTOFF / (NG - 1)
  off = lax.broadcasted_iota(
      jnp.int32, (1, 1, NGP), 2).astype(jnp.float32) * width
  t = (d - off) * (1.0 / width)
  rbf = jnp.exp(-0.5 * t * t).reshape(NB * K, NGP)

  y = jnp.dot(rbf.astype(jnp.bfloat16), w1_ref[...].astype(jnp.bfloat16),
              preferred_element_type=jnp.float32) + b1_ref[...]
  y = _silu(y)
  w = jnp.dot(y.astype(jnp.bfloat16), w2_ref[...].astype(jnp.bfloat16),
              preferred_element_type=jnp.float32) + b2_ref[...]
  m3 = w.reshape(NB, K, HID) * cw_ref[...][:, :, None] * x[:, None, :]
  msg_ref[...] = m3.reshape(NB * K, HID)


def _t1(first, h_or_z, a0, a1, d, cw, prevW, prevb, w1, b1, w2, b2, l1w, l1b):
  grid = N // NB
  full = lambda shape: pl.BlockSpec(shape, lambda i: tuple(0 for _ in shape))
  blk = lambda shape: pl.BlockSpec(shape, lambda i: (i,) + tuple(0 for _ in shape[1:]))
  hspec = blk((NB, 1)) if first else blk((NB, HID))
  aspec = full((1, HID)) if first else blk((NB, HID))
  return pl.pallas_call(
      functools.partial(_t1_body, first),
      grid=(grid,),
      in_specs=[
          hspec, aspec, aspec,
          blk((NB, K)), blk((NB, K)),
          full(prevW.shape), full(prevb.shape),
          full(w1.shape), full(b1.shape), full(w2.shape), full(b2.shape),
          full(l1w.shape), full(l1b.shape),
      ],
      out_specs=[
          blk((NB * K, HID)),
          blk((NB, HID)),
      ],
      out_shape=[
          jax.ShapeDtypeStruct((E, HID), jnp.float32),
          jax.ShapeDtypeStruct((N, HID), jnp.float32),
      ],
  )(h_or_z, a0, a1, d, cw, prevW, prevb, w1, b1, w2, b2, l1w, l1b)


NCHUNK = E // 128
CPS = NCHUNK // 2
CPT = CPS // 16
RPT = N // 16


def _sc_scatter(msg2d, nbr2d):
  mesh = plsc.VectorSubcoreMesh(core_axis_name="c", subcore_axis_name="s")

  @functools.partial(
      pl.kernel,
      out_type=jax.ShapeDtypeStruct((2, N, HID), jnp.float32),
      mesh=mesh,
      scratch_types=[
          pltpu.VMEM((CPT, 128), jnp.int32),
          pltpu.VMEM((128, HID), jnp.float32),
          pltpu.VMEM((128, HID), jnp.float32),
          pltpu.VMEM_SHARED((N, HID), jnp.float32),
          pltpu.SemaphoreType.DMA,
          pltpu.SemaphoreType.DMA,
      ],
  )
  def sc_kernel(msg_hbm, nbr_hbm, out_hbm, idx_v, buf0, buf1, agg_sh,
                sem0, sem1):
    c = lax.axis_index("c")
    s = lax.axis_index("s")

    def zrow(r, _):
      for q in range(HID // 16):
        buf0[r, pl.ds(q * 16, 16)] = jnp.zeros((16,), jnp.float32)
      return 0
    lax.fori_loop(0, 128, zrow, 0)

    def zcopy(q, _):
      pltpu.sync_copy(buf0.at[pl.ds(0, 128)],
                      agg_sh.at[pl.ds(s * RPT + q * 128, 128)])
      return 0
    lax.fori_loop(0, RPT // 128, zcopy, 0)
    plsc.subcore_barrier()

    base = c * CPS + s * CPT

    def gstart(ch, buf, sem):
      pltpu.async_copy(msg_hbm.at[pl.ds((base + ch) * 128, 128)], buf, sem)

    def gwait(ch, buf, sem):
      pltpu.make_async_copy(
          msg_hbm.at[pl.ds((base + ch) * 128, 128)], buf, sem).wait()

    pltpu.sync_copy(nbr_hbm.at[pl.ds(base, CPT)], idx_v)
    gstart(0, buf0, sem0)

    def body(k2, _):
      ca = 2 * k2
      gwait(ca, buf0, sem0)
      gstart(ca + 1, buf1, sem1)
      pltpu.sync_copy(buf0, agg_sh.at[idx_v.at[ca]], add=True)
      gwait(ca + 1, buf1, sem1)

      @pl.when(k2 + 1 < CPT // 2)
      def _():
        gstart(ca + 2, buf0, sem0)

      pltpu.sync_copy(buf1, agg_sh.at[idx_v.at[ca + 1]], add=True)
      return 0

    lax.fori_loop(0, CPT // 2, body, 0)
    plsc.subcore_barrier()

    def wcopy(q, _):
      rows = pl.ds(s * RPT + q * 128, 128)
      pltpu.sync_copy(agg_sh.at[rows], out_hbm.at[c, rows])
      return 0
    lax.fori_loop(0, RPT // 128, wcopy, 0)

  return sc_kernel(msg2d, nbr2d)


def _tf_body(h_ref, a0_ref, a1_ref, batch_ref, l2w_ref, l2b_ref,
             o1w_ref, o1b_ref, o2w_ref, o2b_ref, out_ref):
  agg = a0_ref[...] + a1_ref[...]
  u = jnp.dot(agg, l2w_ref[...], preferred_element_type=jnp.float32) + l2b_ref[...]
  h = h_ref[...] + _silu(u)
  y = _silu(jnp.dot(h, o1w_ref[...], preferred_element_type=jnp.float32) + o1b_ref[...])
  o = jnp.sum(y * o2w_ref[...], axis=1, keepdims=True) + o2b_ref[...]
  oh = (batch_ref[...] == lax.broadcasted_iota(jnp.int32, (N, NGRAPH), 1))
  out_ref[...] = jnp.sum(jnp.where(oh, o, 0.0), axis=0, keepdims=True)


def _tf(h, a0, a1, batch2, l2w, l2b, o1w, o1b, o2w, o2b):
  full = lambda a: pl.BlockSpec(a.shape, lambda: tuple(0 for _ in a.shape))
  return pl.pallas_call(
      _tf_body,
      in_specs=[full(h), full(a0), full(a1), full(batch2), full(l2w),
                full(l2b), full(o1w), full(o1b), full(o2w), full(o2b)],
      out_specs=pl.BlockSpec((1, NGRAPH), lambda: (0, 0)),
      out_shape=jax.ShapeDtypeStruct((1, NGRAPH), jnp.float32),
  )(h, a0, a1, batch2, l2w, l2b, o1w, o1b, o2w, o2b)


def _pad_rows(w, rows):
  return jnp.concatenate([w, jnp.zeros((rows - w.shape[0],) + w.shape[1:], w.dtype)], 0)


def kernel(z, pos, batch, params):
  z = z.astype(jnp.int32)
  batch = batch.astype(jnp.int32)

  gids = jnp.arange(NGRAPH, dtype=jnp.int32)
  seg_lo = jnp.searchsorted(batch, gids, side="left").astype(jnp.int32)
  seg_hi = jnp.searchsorted(batch, gids, side="right").astype(jnp.int32)
  first_rows = batch[jnp.arange(N // RB, dtype=jnp.int32) * RB]
  last_rows = batch[jnp.arange(N // RB, dtype=jnp.int32) * RB + (RB - 1)]
  c0s = seg_lo[first_rows]
  c1s = seg_hi[last_rows]

  nbr, d, cw = _neighbor_search(pos, batch, c0s, c1s)
  nbr2d = nbr.reshape(NCHUNK, 128)

  emb_p = _pad_rows(params["emb"], HID)
  inter = params["inter"]

  h = z.reshape(N, 1)
  a0 = jnp.zeros((1, HID), jnp.float32)
  a1 = a0
  for li, p in enumerate(inter):
    if li == 0:
      prevW, prevb = emb_p, jnp.zeros((1, HID), jnp.float32)
    else:
      q = inter[li - 1]
      prevW, prevb = q["lin2"]["W"].T, q["lin2"]["b"].reshape(1, HID)
    w1 = _pad_rows(p["mlp1"]["W"].T, NGP)
    b1 = p["mlp1"]["b"].reshape(1, HID)
    w2 = p["mlp2"]["W"].T
    b2 = p["mlp2"]["b"].reshape(1, HID)
    l1w = p["lin1"]["W"].T
    l1b = p["lin1"]["b"].reshape(1, HID)
    msg, h = _t1(li == 0, h, a0, a1, d, cw, prevW, prevb, w1, b1, w2, b2,
                 l1w, l1b)
    aggs = _sc_scatter(msg, nbr2d)
    a0, a1 = aggs[0], aggs[1]

  q = inter[-1]
  l2w, l2b = q["lin2"]["W"].T, q["lin2"]["b"].reshape(1, HID)
  o1w = jnp.concatenate([params["out1"]["W"].T,
                         jnp.zeros((HID, HID // 2), jnp.float32)], axis=1)
  o1b = jnp.concatenate([params["out1"]["b"],
                         jnp.zeros((HID // 2,), jnp.float32)]).reshape(1, HID)
  o2w = jnp.concatenate([params["out2"]["W"][0],
                         jnp.zeros((HID // 2,), jnp.float32)]).reshape(1, HID)
  o2b = params["out2"]["b"].reshape(1, 1)
  res = _tf(h, a0, a1, batch.reshape(N, 1), l2w, l2b, o1w, o1b, o2w, o2b)
  return res.reshape(NGRAPH)
